# Initial kernel scaffold; baseline (speedup 1.0000x reference)
import jax
import jax.numpy as jnp
from jax.experimental import pallas as pl


def kernel(nodes, l0_eW0, l0_eb0, l0_eW1, l0_eb1, l0_nW0, l0_nb0, l0_nW1, l0_nb1, l0_nng, l0_nnb, l0_eng, l0_enb, l1_eW0, l1_eb0, l1_eW1, l1_eb1, l1_nW0, l1_nb0, l1_nW1, l1_nb1, l1_nng, l1_nnb):
    raise NotImplementedError("write your pallas kernel here")


# speedup vs baseline: 2.3683x; 2.3683x over previous
import numpy as np
import jax
import jax.numpy as jnp
from jax.experimental import pallas as pl
from jax.experimental.pallas import tpu as pltpu

_B = 128
_N = 60
_NP = 64
_D = 64
_E = _N * (_N - 1)

_SRC = np.repeat(np.arange(_N, dtype=np.int32), _N - 1)
_DST = np.array([j for i in range(_N) for j in range(_N) if j != i],
                dtype=np.int32)


def _ik(nodes_ref,
        e0a, e0b, e0b0, e0W1, e0b1,
        n0a, n0b, n0b0, n0W1, n0b1, n0g, n0bt,
        engG, enbG,
        e1a, e1b, e1c, e1b0, e1W1, e1b1,
        n1a, n1b, n1b0, n1W1, n1b1, n1g, n1bt,
        out_ref):
    x = nodes_ref[0]

    def mm(a, w):
        return jnp.dot(a, w[...], preferred_element_type=jnp.float32)

    eids = jax.lax.broadcasted_iota(jnp.int32, (_NP * _NP, 1), 0)
    s_id = eids // _NP
    d_id = eids % _NP
    emask = ((s_id < _N) & (d_id < _N) & (s_id != d_id)).astype(jnp.float32)
    rmask = (jax.lax.broadcasted_iota(jnp.int32, (_NP, 1), 0) < _N
             ).astype(jnp.float32)
    ecnt = float(_E * _D)
    ncnt = float(_N * _D)

    A = mm(x, e0a)
    Bm = mm(x, e0b)
    h = jnp.maximum(A[:, None, :] + Bm[None, :, :] + e0b0[...], 0.0)
    h = h.reshape(_NP * _NP, _D)
    e = jnp.maximum(mm(h, e0W1) + e0b1[...], 0.0)

    em = e * emask
    agg = em.reshape(_NP, _NP, _D).sum(axis=0)

    hn = jnp.maximum(mm(x, n0a) + mm(agg, n0b) + n0b0[...], 0.0)
    x1 = jnp.maximum(mm(hn, n0W1) + n0b1[...], 0.0)
    mu = jnp.sum(x1 * rmask) / ncnt
    var = jnp.sum((x1 - mu) * (x1 - mu) * rmask) / ncnt
    x1 = (x1 - mu) * jax.lax.rsqrt(var + 1e-5) * n0g[...] + n0bt[...]

    emu = jnp.sum(em) / ecnt
    evar = jnp.sum((e - emu) * (e - emu) * emask) / ecnt
    eln = (e - emu) * jax.lax.rsqrt(evar + 1e-5) * engG[...] + enbG[...]

    A1 = mm(x1, e1a)
    B1 = mm(x1, e1b)
    Ce = mm(eln, e1c)
    h2 = (A1[:, None, :] + B1[None, :, :]).reshape(_NP * _NP, _D)
    h2 = jnp.maximum(h2 + Ce + e1b0[...], 0.0)
    e2 = jnp.maximum(mm(h2, e1W1) + e1b1[...], 0.0)
    agg2 = (e2 * emask).reshape(_NP, _NP, _D).sum(axis=0)

    hn2 = jnp.maximum(mm(x1, n1a) + mm(agg2, n1b) + n1b0[...], 0.0)
    x2 = jnp.maximum(mm(hn2, n1W1) + n1b1[...], 0.0)
    mu2 = jnp.sum(x2 * rmask) / ncnt
    var2 = jnp.sum((x2 - mu2) * (x2 - mu2) * rmask) / ncnt
    out_ref[0] = (x2 - mu2) * jax.lax.rsqrt(var2 + 1e-5) * n1g[...] + n1bt[...]


def kernel(nodes, l0_eW0, l0_eb0, l0_eW1, l0_eb1, l0_nW0, l0_nb0, l0_nW1,
           l0_nb1, l0_nng, l0_nnb, l0_eng, l0_enb, l1_eW0, l1_eb0, l1_eW1,
           l1_eb1, l1_nW0, l1_nb0, l1_nW1, l1_nb1, l1_nng, l1_nnb):
    D, N, NP = _D, _N, _NP
    pad_n = [(0, 0), (0, NP - N), (0, 0)]

    xp = jnp.pad(nodes, pad_n)

    def row(v):
        return v.reshape(1, D)

    def ngrid(v):
        return jnp.pad(v[0], pad_n[1:])

    def egrid(v):
        g = jnp.zeros((NP, NP, D), jnp.float32).at[_SRC, _DST].set(v[0])
        return g.reshape(NP * NP, D)

    ops = [
        l0_eW0[:D], l0_eW0[D:], row(l0_eb0), l0_eW1, row(l0_eb1),
        l0_nW0[:D], l0_nW0[D:], row(l0_nb0), l0_nW1, row(l0_nb1),
        ngrid(l0_nng), ngrid(l0_nnb),
        egrid(l0_eng), egrid(l0_enb),
        l1_eW0[:D], l1_eW0[D:2 * D], l1_eW0[2 * D:], row(l1_eb0),
        l1_eW1, row(l1_eb1),
        l1_nW0[:D], l1_nW0[D:], row(l1_nb0), l1_nW1, row(l1_nb1),
        ngrid(l1_nng), ngrid(l1_nnb),
    ]

    in_specs = [pl.BlockSpec((1, NP, D), lambda i: (i, 0, 0))]
    for o in ops:
        in_specs.append(pl.BlockSpec(o.shape, lambda i, nd=o.ndim: (0,) * nd))

    out = pl.pallas_call(
        _ik,
        grid=(_B,),
        in_specs=in_specs,
        out_specs=pl.BlockSpec((1, NP, D), lambda i: (i, 0, 0)),
        out_shape=jax.ShapeDtypeStruct((_B, NP, D), jnp.float32),
        compiler_params=pltpu.CompilerParams(
            dimension_semantics=("arbitrary",)),
    )(xp, *ops)
    return out[:, :N, :]
